# baseline (speedup 1.0000x reference)
import jax
import jax.numpy as jnp
from jax.experimental import pallas as pl
from jax.experimental.pallas import tpu as pltpu
from jax.experimental.pallas import tpu_sc as plsc

N_HEADS = 16
D = 64
WINDOW = 400
GRID = 4
CHUNK = 1250


def _sc_gather(table2, idx2):
    n_idx = idx2.shape[0]
    mesh = plsc.VectorSubcoreMesh(core_axis_name="core", subcore_axis_name="subcore")

    @pl.kernel(
        out_type=jax.ShapeDtypeStruct((n_idx, 2 * D), table2.dtype),
        mesh=mesh,
    )
    def kern(x_hbm, i_hbm, o_hbm):
        def body(i_vmem, o_vmem):
            pltpu.sync_copy(x_hbm.at[i_vmem.at[0, 0]], o_vmem)

        pltpu.emit_pipeline(
            body,
            grid=(n_idx // WINDOW,),
            in_specs=[pl.BlockSpec((1, 1, WINDOW), index_map=lambda i: (i, 0, 0))],
            out_specs=[pl.BlockSpec((WINDOW, 2 * D), index_map=lambda i: (i, 0))],
            core_axis_name=("core", "subcore"),
            dimension_semantics=(pltpu.PARALLEL,),
        )(i_hbm, o_hbm)

    return kern(table2, idx2.reshape(n_idx // WINDOW, 1, WINDOW))


def _tc_select_broadcast(g2, par2, heads):
    n = g2.shape[0]
    rows = n // GRID

    def body(g_ref, p_ref, out_hbm, sel_ref, sems):
        c = pl.program_id(0)
        slot = c % 2

        @pl.when(c >= 2)
        def _():
            for a in range(heads):
                pltpu.make_async_copy(
                    sel_ref.at[slot],
                    out_hbm.at[a, pl.ds((c - 2) * rows, rows)],
                    sems.at[slot, a],
                ).wait()

        for k in range(rows // CHUNK):
            sl = pl.ds(k * CHUNK, CHUNK)
            g = g_ref[sl]
            pe = p_ref[sl, 0:1]
            po = p_ref[sl, 8:9]
            a0 = g[:, :D]
            a1 = g[:, D : 2 * D]
            b0 = g[:, 2 * D : 3 * D]
            b1 = g[:, 3 * D :]
            sel_ref[slot, sl] = jnp.concatenate(
                [a0 + (a1 - a0) * pe, b0 + (b1 - b0) * po], axis=1
            )

        for a in range(heads):
            pltpu.make_async_copy(
                sel_ref.at[slot],
                out_hbm.at[a, pl.ds(c * rows, rows)],
                sems.at[slot, a],
            ).start()

        @pl.when(c == GRID - 1)
        def _():
            for a in range(heads):
                pltpu.make_async_copy(
                    sel_ref.at[1 - slot],
                    out_hbm.at[a, pl.ds((c - 1) * rows, rows)],
                    sems.at[1 - slot, a],
                ).wait()
                pltpu.make_async_copy(
                    sel_ref.at[slot],
                    out_hbm.at[a, pl.ds(c * rows, rows)],
                    sems.at[slot, a],
                ).wait()

    return pl.pallas_call(
        body,
        grid=(GRID,),
        in_specs=[
            pl.BlockSpec((rows, 4 * D), lambda c: (c, 0)),
            pl.BlockSpec((rows, 16), lambda c: (c, 0)),
        ],
        out_specs=pl.BlockSpec(memory_space=pl.ANY),
        out_shape=jax.ShapeDtypeStruct((heads, n, 2 * D), g2.dtype),
        scratch_shapes=[
            pltpu.VMEM((2, rows, 2 * D), g2.dtype),
            pltpu.SemaphoreType.DMA((2, heads)),
        ],
    )(g2, par2)


def kernel(src_seq, structure_emb):
    batch, num_posts, _ = src_seq.shape
    m = num_posts * num_posts
    n = batch * m // 2
    flat_idx = src_seq.reshape(-1).astype(jnp.int32)
    table2 = structure_emb.reshape(-1, 2 * D)
    gathered = _sc_gather(table2, flat_idx >> 1)
    g2 = gathered.reshape(n, 4 * D)
    par = (flat_idx & 1).astype(jnp.float32)
    par2 = jnp.repeat(par.reshape(n, 2), 8, axis=1)
    out = _tc_select_broadcast(g2, par2, N_HEADS)
    return out.reshape(batch, N_HEADS, num_posts, num_posts, D)


# speedup vs baseline: 1.0346x; 1.0346x over previous
import jax
import jax.numpy as jnp
from jax.experimental import pallas as pl
from jax.experimental.pallas import tpu as pltpu
from jax.experimental.pallas import tpu_sc as plsc

N_HEADS = 16
D = 64
WINDOW = 400


def _sc_gather(table2, idx2):
    n_idx = idx2.shape[0]
    mesh = plsc.VectorSubcoreMesh(core_axis_name="core", subcore_axis_name="subcore")

    @pl.kernel(
        out_type=jax.ShapeDtypeStruct((n_idx, 2 * D), table2.dtype),
        mesh=mesh,
    )
    def kern(x_hbm, i_hbm, o_hbm):
        def body(i_vmem, o_vmem):
            pltpu.sync_copy(x_hbm.at[i_vmem.at[0, 0]], o_vmem)

        pltpu.emit_pipeline(
            body,
            grid=(n_idx // WINDOW,),
            in_specs=[pl.BlockSpec((1, 1, WINDOW), index_map=lambda i: (i, 0, 0))],
            out_specs=[pl.BlockSpec((WINDOW, 2 * D), index_map=lambda i: (i, 0))],
            core_axis_name=("core", "subcore"),
            dimension_semantics=(pltpu.PARALLEL,),
        )(i_hbm, o_hbm)

    return kern(table2, idx2.reshape(n_idx // WINDOW, 1, WINDOW))


def _tc_pack(table, rows_per_step=2000):
    n = table.shape[0]
    half = rows_per_step // 2

    def body(x_ref, o_ref):
        x = x_ref[...]
        o_ref[...] = jnp.concatenate([x[:half], x[half:]], axis=1)

    return pl.pallas_call(
        body,
        grid=(n // rows_per_step,),
        in_specs=[pl.BlockSpec((rows_per_step, D), lambda c: (c, 0))],
        out_specs=pl.BlockSpec((rows_per_step // 2, 2 * D), lambda c: (c, 0)),
        out_shape=jax.ShapeDtypeStruct((n // 2, 2 * D), table.dtype),
    )(table)


def _tc_select_broadcast(g3, par3, batch, heads, mh):
    def body(g_ref, p_ref, out_hbm, sel_ref, sems):
        c = pl.program_id(0)
        g = g_ref[0].reshape(mh, 4 * D)
        pe = p_ref[0][:, 0:1]
        po = p_ref[0][:, 8:9]
        a0 = g[:, :D]
        a1 = g[:, D : 2 * D]
        b0 = g[:, 2 * D : 3 * D]
        b1 = g[:, 3 * D :]
        sel_ref[...] = jnp.concatenate(
            [a0 + (a1 - a0) * pe, b0 + (b1 - b0) * po], axis=1
        )
        copies = [
            pltpu.make_async_copy(sel_ref, out_hbm.at[a, c], sems.at[a])
            for a in range(heads)
        ]
        for cp in copies:
            cp.start()
        for cp in copies:
            cp.wait()

    return pl.pallas_call(
        body,
        grid=(batch,),
        in_specs=[
            pl.BlockSpec((1, 2 * mh, 2 * D), lambda c: (c, 0, 0)),
            pl.BlockSpec((1, mh, 16), lambda c: (c, 0, 0)),
        ],
        out_specs=pl.BlockSpec(memory_space=pl.ANY),
        out_shape=jax.ShapeDtypeStruct((heads, batch, mh, 2 * D), g3.dtype),
        scratch_shapes=[
            pltpu.VMEM((mh, 2 * D), g3.dtype),
            pltpu.SemaphoreType.DMA((heads,)),
        ],
    )(g3, par3)


def kernel(src_seq, structure_emb):
    batch, num_posts, _ = src_seq.shape
    m = num_posts * num_posts
    mh = m // 2
    flat_idx = src_seq.reshape(-1).astype(jnp.int32)
    table2 = _tc_pack(structure_emb)
    packed_row = (flat_idx // 2000) * 1000 + flat_idx % 1000
    half_sel = (flat_idx // 1000) & 1
    gathered = _sc_gather(table2, packed_row)
    g3 = gathered.reshape(batch, m, 2 * D)
    par = half_sel.astype(jnp.float32)
    par3 = jnp.repeat(par.reshape(batch, mh, 2), 8, axis=2)
    out = _tc_select_broadcast(g3, par3, batch, N_HEADS, mh)
    return out.reshape(batch, N_HEADS, num_posts, num_posts, D)
